# Initial kernel scaffold; baseline (speedup 1.0000x reference)
import jax
import jax.numpy as jnp
from jax.experimental import pallas as pl


def kernel(x, edge_index, batch, W_lin1, b_lin1, W_agg1, b_agg1, W_lin2, b_lin2, W_agg2, b_agg2, W_post1, b_post1, W_post2, b_post2):
    raise NotImplementedError("write your pallas kernel here")


# speedup vs baseline: 5.7373x; 5.7373x over previous
import functools

import jax
import jax.numpy as jnp
from jax import lax
from jax.experimental import pallas as pl
from jax.experimental.pallas import tpu as pltpu
from jax.experimental.pallas import tpu_sc as plsc

N_NODES = 10000
N_EDGES = 320000
D = 128
N_GRAPHS = 64
D_OUT = 16

NC = 2
NS = 16
NW = NC * NS
EC = 128
N_CHUNKS = N_EDGES // EC
CHUNKS_PER_W = -(-N_CHUNKS // NW)
N_ROWCH = -(-N_NODES // EC)
N_PAD = N_ROWCH * EC

BLK = 1000
GRID = N_NODES // BLK


def _sc_segment_pass(with_deg: bool):
  mesh = plsc.VectorSubcoreMesh(core_axis_name="c", subcore_axis_name="s")
  out_type = [jax.ShapeDtypeStruct((NC, N_PAD, D), jnp.float32)]
  scratch = [
      pltpu.VMEM((EC,), jnp.int32),
      pltpu.VMEM((EC,), jnp.int32),
      pltpu.VMEM((EC, D), jnp.float32),
      pltpu.VMEM_SHARED((N_PAD, D), jnp.float32),
      pltpu.SemaphoreType.DMA,
  ]
  if with_deg:
    out_type.append(jax.ShapeDtypeStruct((NC, N_PAD), jnp.float32))
    scratch += [
        pltpu.VMEM((EC,), jnp.float32),
        pltpu.VMEM((EC,), jnp.float32),
        pltpu.VMEM_SHARED((N_PAD,), jnp.float32),
    ]

  def body(y_hbm, src_hbm, dst_hbm, zeros_hbm, ones_hbm, *rest):
    if with_deg:
      (out_hbm, deg_hbm, src_v, dst_v, rows_v, acc_sh, sem,
       ones_v, zcol_v, deg_sh) = rest
    else:
      (out_hbm, src_v, dst_v, rows_v, acc_sh, sem) = rest
    cid = lax.axis_index("c")
    sid = lax.axis_index("s")
    wid = sid * NC + cid

    pltpu.sync_copy(zeros_hbm, rows_v)
    if with_deg:
      pltpu.sync_copy(ones_hbm, ones_v)
      pltpu.sync_copy(zeros_hbm.at[0, :], zcol_v)
    for k in range((N_ROWCH + NS - 1) // NS):
      r = sid + k * NS

      @pl.when(r < N_ROWCH)
      def _zero():
        pltpu.sync_copy(rows_v, acc_sh.at[pl.ds(r * EC, EC), :])
        if with_deg:
          pltpu.sync_copy(zcol_v, deg_sh.at[pl.ds(r * EC, EC)])

    plsc.subcore_barrier()

    def edge_step(k, carry):
      chunk = wid + k * NW

      @pl.when(chunk < N_CHUNKS)
      def _():
        off = chunk * EC
        pltpu.sync_copy(src_hbm.at[pl.ds(off, EC)], src_v)
        pltpu.sync_copy(dst_hbm.at[pl.ds(off, EC)], dst_v)
        pltpu.async_copy(y_hbm.at[src_v], rows_v, sem).wait()
        pltpu.sync_copy(rows_v, acc_sh.at[dst_v], add=True)
        if with_deg:
          pltpu.sync_copy(ones_v, deg_sh.at[dst_v], add=True)

      return carry

    lax.fori_loop(0, CHUNKS_PER_W, edge_step, 0)
    plsc.subcore_barrier()

    for k in range((N_ROWCH + NS - 1) // NS):
      r = sid + k * NS

      @pl.when(r < N_ROWCH)
      def _wb():
        pltpu.sync_copy(acc_sh.at[pl.ds(r * EC, EC), :], rows_v)
        pltpu.sync_copy(rows_v, out_hbm.at[cid, pl.ds(r * EC, EC), :])
        if with_deg:
          pltpu.sync_copy(deg_sh.at[pl.ds(r * EC, EC)], ones_v)
          pltpu.sync_copy(ones_v, deg_hbm.at[cid, pl.ds(r * EC, EC)])

  return pl.kernel(
      body,
      out_type=tuple(out_type) if with_deg else out_type[0],
      mesh=mesh,
      scratch_types=scratch,
  )


def _lin_relu_kernel(x_ref, w_ref, b_ref, o_ref):
  o_ref[...] = jax.nn.relu(
      jnp.dot(x_ref[...], w_ref[...], preferred_element_type=jnp.float32)
      + b_ref[...])


def _node_linear(x, w, b):
  return pl.pallas_call(
      _lin_relu_kernel,
      grid=(GRID,),
      in_specs=[
          pl.BlockSpec((BLK, D), lambda i: (i, 0)),
          pl.BlockSpec((D, D), lambda i: (0, 0)),
          pl.BlockSpec((1, D), lambda i: (0, 0)),
      ],
      out_specs=pl.BlockSpec((BLK, D), lambda i: (i, 0)),
      out_shape=jax.ShapeDtypeStruct((N_NODES, D), jnp.float32),
  )(x, w, b.reshape(1, D))


def _combine_block(p_ref, deg_ref, x_ref, wt_ref, wb_ref, b_ref, i):
  psum = p_ref[0] + p_ref[1]
  dsum = deg_ref[0, 0, 0, :] + deg_ref[1, 0, 0, :]
  inv = 1.0 / jnp.maximum(dsum, 1.0)
  aggr = psum * inv[:, None]
  h = jax.nn.relu(
      jnp.dot(aggr, wt_ref[...], preferred_element_type=jnp.float32)
      + jnp.dot(x_ref[...], wb_ref[...], preferred_element_type=jnp.float32)
      + b_ref[...])
  nrm = jnp.sqrt(jnp.sum(h * h, axis=1, keepdims=True))
  return h / jnp.maximum(nrm, 1e-12)


def _combine2_kernel(p_ref, deg_ref, x_ref, wt_ref, wb_ref, b_ref,
                     w2_ref, b2_ref, h_ref, y2_ref):
  i = pl.program_id(0)
  h = _combine_block(p_ref, deg_ref, x_ref, wt_ref, wb_ref, b_ref, i)
  h_ref[...] = h
  y2_ref[...] = jax.nn.relu(
      jnp.dot(h, w2_ref[...], preferred_element_type=jnp.float32)
      + b2_ref[...])


def _combine_and_next(p, deg, x, w_agg, b_agg, w_lin2, b_lin2):
  wt, wb = w_agg[:D], w_agg[D:]
  return pl.pallas_call(
      _combine2_kernel,
      grid=(GRID,),
      in_specs=[
          pl.BlockSpec((NC, BLK, D), lambda i: (0, i, 0)),
          pl.BlockSpec((NC, 1, 1, BLK), lambda i: (0, i, 0, 0)),
          pl.BlockSpec((BLK, D), lambda i: (i, 0)),
          pl.BlockSpec((D, D), lambda i: (0, 0)),
          pl.BlockSpec((D, D), lambda i: (0, 0)),
          pl.BlockSpec((1, D), lambda i: (0, 0)),
          pl.BlockSpec((D, D), lambda i: (0, 0)),
          pl.BlockSpec((1, D), lambda i: (0, 0)),
      ],
      out_specs=[
          pl.BlockSpec((BLK, D), lambda i: (i, 0)),
          pl.BlockSpec((BLK, D), lambda i: (i, 0)),
      ],
      out_shape=[
          jax.ShapeDtypeStruct((N_NODES, D), jnp.float32),
          jax.ShapeDtypeStruct((N_NODES, D), jnp.float32),
      ],
  )(p, deg.reshape(NC, GRID, 1, BLK), x, wt, wb, b_agg.reshape(1, D),
    w_lin2, b_lin2.reshape(1, D))


def _final_kernel(p_ref, deg_ref, x_ref, wt_ref, wb_ref, b_ref,
                  batch_ref, wp1_ref, bp1_ref, wp2_ref, bp2_ref,
                  o_ref, pool_ref):
  i = pl.program_id(0)
  h = _combine_block(p_ref, deg_ref, x_ref, wt_ref, wb_ref, b_ref, i)
  ids_col = batch_ref[0]

  @pl.when(i == 0)
  def _init():
    pool_ref[...] = jnp.full((N_GRAPHS, D), -jnp.inf, jnp.float32)

  neg = jnp.float32(-jnp.inf)
  parts = []
  for g in range(N_GRAPHS):
    m = ids_col == g
    parts.append(jnp.max(jnp.where(m, h, neg), axis=0))
  stacked = jnp.stack(parts, axis=0)
  pool_ref[...] = jnp.maximum(pool_ref[...], stacked)

  @pl.when(i == GRID - 1)
  def _fin():
    pooled = pool_ref[...]
    o1 = jnp.dot(pooled, wp1_ref[...],
                 preferred_element_type=jnp.float32) + bp1_ref[...]
    o2 = jnp.dot(o1, wp2_ref[...],
                 preferred_element_type=jnp.float32) + bp2_ref[...]
    z = o2 - jnp.max(o2, axis=1, keepdims=True)
    o_ref[...] = z - jnp.log(jnp.sum(jnp.exp(z), axis=1, keepdims=True))


def _final_stage(q, deg, h1, w_agg, b_agg, batch, wp1, bp1, wp2, bp2):
  wt, wb = w_agg[:D], w_agg[D:]
  return pl.pallas_call(
      _final_kernel,
      grid=(GRID,),
      in_specs=[
          pl.BlockSpec((NC, BLK, D), lambda i: (0, i, 0)),
          pl.BlockSpec((NC, 1, 1, BLK), lambda i: (0, i, 0, 0)),
          pl.BlockSpec((BLK, D), lambda i: (i, 0)),
          pl.BlockSpec((D, D), lambda i: (0, 0)),
          pl.BlockSpec((D, D), lambda i: (0, 0)),
          pl.BlockSpec((1, D), lambda i: (0, 0)),
          pl.BlockSpec((1, BLK, 1), lambda i: (i, 0, 0)),
          pl.BlockSpec((D, D), lambda i: (0, 0)),
          pl.BlockSpec((1, D), lambda i: (0, 0)),
          pl.BlockSpec((D, D_OUT), lambda i: (0, 0)),
          pl.BlockSpec((1, D_OUT), lambda i: (0, 0)),
      ],
      out_specs=pl.BlockSpec((N_GRAPHS, D_OUT), lambda i: (0, 0)),
      out_shape=jax.ShapeDtypeStruct((N_GRAPHS, D_OUT), jnp.float32),
      scratch_shapes=[pltpu.VMEM((N_GRAPHS, D), jnp.float32)],
  )(q, deg.reshape(NC, GRID, 1, BLK), h1, wt, wb, b_agg.reshape(1, D),
    batch.reshape(GRID, BLK, 1), wp1, bp1.reshape(1, D),
    wp2, bp2.reshape(1, D_OUT))


def kernel(x, edge_index, batch,
           W_lin1, b_lin1, W_agg1, b_agg1,
           W_lin2, b_lin2, W_agg2, b_agg2,
           W_post1, b_post1, W_post2, b_post2):
  src = edge_index[0]
  dst = edge_index[1]
  zeros = jnp.zeros((EC, D), jnp.float32)
  ones = jnp.ones((EC,), jnp.float32)

  y1 = _node_linear(x, W_lin1, b_lin1)
  p1, deg = _sc_segment_pass(True)(y1, src, dst, zeros, ones)
  p1, deg = p1[:, :N_NODES], deg[:, :N_NODES]
  h1, y2 = _combine_and_next(p1, deg, x, W_agg1, b_agg1, W_lin2, b_lin2)
  p2 = _sc_segment_pass(False)(y2, src, dst, zeros, ones)
  return _final_stage(p2[:, :N_NODES], deg, h1, W_agg2, b_agg2, batch,
                      W_post1, b_post1, W_post2, b_post2)
